# baseline (speedup 1.0000x reference)
import functools

import jax
import jax.numpy as jnp
from jax import lax
from jax.experimental import pallas as pl
from jax.experimental.pallas import tpu as pltpu
from jax.experimental.pallas import tpu_sc as plsc

N = 50000
E = 800000
EP = 819200
CHUNK = 64
ROWS = EP // CHUNK
H = 64
HALF = 32
NC, NS, L = 2, 16, 16
NACC = 50048
TRASH = N
BLK = 1000


def _k1_body(x_ref, win_ref, bin_ref, wa_ref, ba_ref, h_ref, p3_ref):
    h = jnp.maximum(jnp.dot(x_ref[...], win_ref[...],
                            preferred_element_type=jnp.float32, precision=lax.Precision.HIGHEST) + bin_ref[...], 0.0)
    h_ref[...] = h
    p = jnp.dot(h, wa_ref[...], preferred_element_type=jnp.float32, precision=lax.Precision.HIGHEST) + ba_ref[...]
    p3_ref[...] = jnp.stack([p[:, :HALF], p[:, HALF:]], axis=0)


def _input_proj(x, W_in, b_in, Wa_h, ba):
    grid = N // BLK
    return pl.pallas_call(
        _k1_body,
        grid=(grid,),
        in_specs=[
            pl.BlockSpec((BLK, 128), lambda r: (r, 0)),
            pl.BlockSpec((128, H), lambda r: (0, 0)),
            pl.BlockSpec((1, H), lambda r: (0, 0)),
            pl.BlockSpec((H, H), lambda r: (0, 0)),
            pl.BlockSpec((1, H), lambda r: (0, 0)),
        ],
        out_specs=[
            pl.BlockSpec((BLK, H), lambda r: (r, 0)),
            pl.BlockSpec((2, BLK, HALF), lambda r: (0, r, 0)),
        ],
        out_shape=[
            jax.ShapeDtypeStruct((N, H), jnp.float32),
            jax.ShapeDtypeStruct((2, N, HALF), jnp.float32),
        ],
    )(x, W_in, b_in, Wa_h, ba)


def _kq_body(ea_ref, w0_ref, w1_ref, w2_ref, q0_ref, q1_ref, q2_ref):
    ea = ea_ref[...]
    for w, out in ((w0_ref, q0_ref), (w1_ref, q1_ref), (w2_ref, q2_ref)):
        out[...] = jnp.dot(ea, w[...], preferred_element_type=jnp.float32,
                           precision=lax.Precision.HIGHEST)


def _edge_proj(ea2, W0e2, W1e2, W2e2):
    eb = 8000
    grid = (E // 2) // eb
    return pl.pallas_call(
        _kq_body,
        grid=(grid,),
        in_specs=[
            pl.BlockSpec((eb, 32), lambda r: (r, 0)),
            pl.BlockSpec((32, 128), lambda r: (0, 0)),
            pl.BlockSpec((32, 128), lambda r: (0, 0)),
            pl.BlockSpec((32, 128), lambda r: (0, 0)),
        ],
        out_specs=[pl.BlockSpec((eb, 128), lambda r: (r, 0))] * 3,
        out_shape=[jax.ShapeDtypeStruct((E // 2, 128), jnp.float32)] * 3,
    )(ea2, W0e2, W1e2, W2e2)


def _kagg_body(acc_ref, degp_ref, wb_ref, bb_ref, agg_ref, stats_ref):
    r = pl.program_id(0)
    s = jnp.concatenate([acc_ref[0], acc_ref[1]], axis=-1)
    deg = (degp_ref[0, :, 0:1] + degp_ref[1, :, 0:1])
    a = jnp.dot(s, wb_ref[...], preferred_element_type=jnp.float32, precision=lax.Precision.HIGHEST) + deg * bb_ref[...]
    agg_ref[...] = a
    zc = jnp.zeros((1, 128 - H), jnp.float32)
    r0 = jnp.concatenate([jnp.sum(a, axis=0).reshape(1, H), zc], axis=1)
    r1 = jnp.concatenate([jnp.sum(a * a, axis=0).reshape(1, H), zc], axis=1)
    upd = jnp.concatenate([r0, r1, jnp.zeros((6, 128), jnp.float32)], axis=0)

    @pl.when(r == 0)
    def _():
        stats_ref[...] = jnp.zeros((8, 128), jnp.float32)

    stats_ref[...] += upd


def _aggregate(acc2, degp, Wb, bb):
    grid = N // BLK
    return pl.pallas_call(
        _kagg_body,
        grid=(grid,),
        in_specs=[
            pl.BlockSpec((2, BLK, HALF), lambda r: (0, r, 0)),
            pl.BlockSpec((2, BLK, 16), lambda r: (0, r, 0)),
            pl.BlockSpec((H, H), lambda r: (0, 0)),
            pl.BlockSpec((1, H), lambda r: (0, 0)),
        ],
        out_specs=[
            pl.BlockSpec((BLK, H), lambda r: (r, 0)),
            pl.BlockSpec((8, 128), lambda r: (0, 0)),
        ],
        out_shape=[
            jax.ShapeDtypeStruct((N, H), jnp.float32),
            jax.ShapeDtypeStruct((8, 128), jnp.float32),
        ],
    )(acc2, degp, Wb, bb)


def _bn_from_stats(stats_ref, agg, g_ref, be_ref, h_ref):
    mu = stats_ref[0:1, :H] * (1.0 / N)
    ex2 = stats_ref[1:2, :H] * (1.0 / N)
    var = ex2 - mu * mu
    inv = g_ref[...] * lax.rsqrt(var + 1e-5)
    return h_ref[...] + jnp.maximum((agg - mu) * inv + be_ref[...], 0.0)


def _kupd_body(agg_ref, stats_ref, g_ref, be_ref, h_ref, wa_ref, ba_ref,
               hn_ref, p3_ref):
    hn = _bn_from_stats(stats_ref, agg_ref[...], g_ref, be_ref, h_ref)
    hn_ref[...] = hn
    p = jnp.dot(hn, wa_ref[...], preferred_element_type=jnp.float32, precision=lax.Precision.HIGHEST) + ba_ref[...]
    p3_ref[...] = jnp.stack([p[:, :HALF], p[:, HALF:]], axis=0)


def _update_and_proj(agg, stats, g, be, h, Wa_h, ba):
    grid = N // BLK
    return pl.pallas_call(
        _kupd_body,
        grid=(grid,),
        in_specs=[
            pl.BlockSpec((BLK, H), lambda r: (r, 0)),
            pl.BlockSpec((8, 128), lambda r: (0, 0)),
            pl.BlockSpec((1, H), lambda r: (0, 0)),
            pl.BlockSpec((1, H), lambda r: (0, 0)),
            pl.BlockSpec((BLK, H), lambda r: (r, 0)),
            pl.BlockSpec((H, H), lambda r: (0, 0)),
            pl.BlockSpec((1, H), lambda r: (0, 0)),
        ],
        out_specs=[
            pl.BlockSpec((BLK, H), lambda r: (r, 0)),
            pl.BlockSpec((2, BLK, HALF), lambda r: (0, r, 0)),
        ],
        out_shape=[
            jax.ShapeDtypeStruct((N, H), jnp.float32),
            jax.ShapeDtypeStruct((2, N, HALF), jnp.float32),
        ],
    )(agg, stats, g, be, h, Wa_h, ba)


def _kout_body(agg_ref, stats_ref, g_ref, be_ref, h_ref,
               wo1_ref, bo1_ref, wo2_ref, bo2_ref, out_ref):
    hn = _bn_from_stats(stats_ref, agg_ref[...], g_ref, be_ref, h_ref)
    t = jnp.maximum(jnp.dot(hn, wo1_ref[...],
                            preferred_element_type=jnp.float32, precision=lax.Precision.HIGHEST) + bo1_ref[...], 0.0)
    out_ref[...] = jnp.dot(t, wo2_ref[...],
                           preferred_element_type=jnp.float32, precision=lax.Precision.HIGHEST) + bo2_ref[...]


def _final_out(agg, stats, g, be, h, Wo1, bo1, Wo2p, bo2p):
    grid = N // BLK
    return pl.pallas_call(
        _kout_body,
        grid=(grid,),
        in_specs=[
            pl.BlockSpec((BLK, H), lambda r: (r, 0)),
            pl.BlockSpec((8, 128), lambda r: (0, 0)),
            pl.BlockSpec((1, H), lambda r: (0, 0)),
            pl.BlockSpec((1, H), lambda r: (0, 0)),
            pl.BlockSpec((BLK, H), lambda r: (r, 0)),
            pl.BlockSpec((H, H), lambda r: (0, 0)),
            pl.BlockSpec((1, H), lambda r: (0, 0)),
            pl.BlockSpec((H, 128), lambda r: (0, 0)),
            pl.BlockSpec((1, 128), lambda r: (0, 0)),
        ],
        out_specs=[pl.BlockSpec((BLK, 128), lambda r: (r, 0))],
        out_shape=[jax.ShapeDtypeStruct((N, 128), jnp.float32)],
    )(agg, stats, g, be, h, Wo1, bo1, Wo2p, bo2p)[0]


_MESH = plsc.VectorSubcoreMesh(core_axis_name="c", subcore_axis_name="s",
                               num_cores=NC, num_subcores=NS)

ZROWS = 136


def _zero_spmem(zbuf, acc_sh, sid, width):
    @pl.loop(0, ZROWS)
    def _(i):
        for c2 in range(width // L):
            zbuf[i, pl.ds(c2 * L, L)] = jnp.zeros((L,), jnp.float32)

    for t in range(23):
        pltpu.sync_copy(zbuf, acc_sh.at[pl.ds(sid * 3128 + t * ZROWS, ZROWS), :])


def _writeback(acc_sh, out, cid, sid):
    nb = 3128
    start = pl.multiple_of(jnp.minimum(sid * nb, N - nb), 8)
    pltpu.sync_copy(acc_sh.at[pl.ds(start, nb), :],
                    out.at[cid, pl.ds(start, nb), :])


CPS = ROWS // NS


def _sc_layer_body(src2, dst2d, p_flat, q_flat, acc_out,
                   acc_sh, sidx, didx, rows, qv, zbuf, isem, gsem, qsem, ssem):
    cid = lax.axis_index("c")
    sid = lax.axis_index("s")
    _zero_spmem(zbuf, acc_sh, sid, HALF)
    plsc.subcore_barrier()

    row0 = sid * CPS
    qcol = cid * HALF

    def drain(src, dst, sem):
        pltpu.make_async_copy(src, dst, sem).wait()

    pltpu.async_copy(src2.at[cid, row0], sidx.at[0], isem.at[0])
    pltpu.async_copy(dst2d.at[row0], didx.at[0], isem.at[0])
    pltpu.async_copy(src2.at[cid, row0 + 1], sidx.at[1], isem.at[1])
    pltpu.async_copy(dst2d.at[row0 + 1], didx.at[1], isem.at[1])

    qdummy = q_flat.at[pl.ds(0, CHUNK), pl.ds(0, HALF)]
    gdummy = p_flat.at[pl.ds(0, CHUNK), :]
    sdummy = acc_sh.at[pl.ds(0, CHUNK), :]

    @pl.loop(0, CPS + 8, step=8)
    def _steps(t0):
        for b in range(8):
            t = t0 + b
            bd = b % 4
            bp = (b + 2) % 4
            ip = (b + 6) % 8
            inx = (b + 2) % 8

            @pl.when(t < CPS)
            def _():
                @pl.when(t >= 4)
                def _():
                    drain(gdummy, sdummy, ssem.at[bd])

                drain(src2.at[cid, 0], sidx.at[b], isem.at[b])
                drain(dst2d.at[0], didx.at[b], isem.at[b])
                pltpu.async_copy(p_flat.at[sidx.at[b]], rows.at[bd], gsem.at[bd])
                qoff = jnp.minimum((row0 + t) * 32, E // 2 - 32)
                pltpu.async_copy(
                    q_flat.at[pl.ds(qoff, 32), pl.ds(qcol, HALF)],
                    qv.at[bd, pl.ds(0, 32), :], qsem.at[bd])
                pltpu.async_copy(
                    q_flat.at[pl.ds(qoff, 32), pl.ds(64 + qcol, HALF)],
                    qv.at[bd, pl.ds(32, 32), :], qsem.at[bd])

            @pl.when(jnp.logical_and(t >= 2, t < CPS + 2))
            def _():
                drain(gdummy, rows.at[bp], gsem.at[bp])
                drain(qdummy, qv.at[bp], qsem.at[bp])

                @pl.loop(0, CHUNK, unroll=4)
                def _cmp(i):
                    for c2 in range(2):
                        sl = pl.ds(c2 * L, L)
                        rows[bp, i, sl] = jnp.maximum(
                            rows[bp, i, sl] + qv[bp, i, sl], 0.0)

                pltpu.async_copy(rows.at[bp], acc_sh.at[didx.at[ip]],
                                 ssem.at[bp], add=True)

            @pl.when(t + 2 < CPS)
            def _():
                pltpu.async_copy(src2.at[cid, row0 + t + 2], sidx.at[inx],
                                 isem.at[inx])
                pltpu.async_copy(dst2d.at[row0 + t + 2], didx.at[inx],
                                 isem.at[inx])

    for slot in range(4):
        drain(gdummy, sdummy, ssem.at[slot])
    plsc.subcore_barrier()
    _writeback(acc_sh, acc_out, cid, sid)


def _sc_layer(src2, dst2d, p_flat, q_flat):
    return pl.kernel(
        _sc_layer_body,
        out_type=jax.ShapeDtypeStruct((2, N, HALF), jnp.float32),
        mesh=_MESH,
        scratch_types=[
            pltpu.VMEM_SHARED((NACC, HALF), jnp.float32),
            pltpu.VMEM((8, CHUNK), jnp.int32),
            pltpu.VMEM((8, CHUNK), jnp.int32),
            pltpu.VMEM((4, CHUNK, HALF), jnp.float32),
            pltpu.VMEM((4, CHUNK, HALF), jnp.float32),
            pltpu.VMEM((ZROWS, HALF), jnp.float32),
            pltpu.SemaphoreType.DMA((8,)),
            pltpu.SemaphoreType.DMA((4,)),
            pltpu.SemaphoreType.DMA((4,)),
            pltpu.SemaphoreType.DMA((4,)),
        ],
        compiler_params=pltpu.CompilerParams(use_tc_tiling_on_sc=False),
    )(src2, dst2d, p_flat, q_flat)


def _sc_deg_body(dst2d, deg_out, acc_sh, didx, ones, zbuf):
    cid = lax.axis_index("c")
    sid = lax.axis_index("s")
    _zero_spmem(zbuf, acc_sh, sid, 16)

    @pl.loop(0, CHUNK)
    def _(i):
        ones[i, pl.ds(0, L)] = jnp.full((L,), 1.0, jnp.float32)

    plsc.subcore_barrier()

    wid = cid * NS + sid
    row0 = wid * (ROWS // (NC * NS))

    @pl.loop(0, ROWS // (NC * NS) // 16)
    def _outer(o):
        r0 = row0 + o * 16
        pltpu.sync_copy(dst2d.at[pl.ds(r0, 16), :], didx)

        @pl.loop(0, 16)
        def _chunk(j):
            pltpu.sync_copy(ones, acc_sh.at[didx.at[j]], add=True)

    plsc.subcore_barrier()
    _writeback(acc_sh, deg_out, cid, sid)


def _sc_degrees(dst2d):
    return pl.kernel(
        _sc_deg_body,
        out_type=jax.ShapeDtypeStruct((2, N, 16), jnp.float32),
        mesh=_MESH,
        scratch_types=[
            pltpu.VMEM_SHARED((NACC, 16), jnp.float32),
            pltpu.VMEM((16, CHUNK), jnp.int32),
            pltpu.VMEM((CHUNK, 16), jnp.float32),
            pltpu.VMEM((ZROWS, 16), jnp.float32),
        ],
        compiler_params=pltpu.CompilerParams(use_tc_tiling_on_sc=False),
    )(dst2d)


def kernel(x, edge_index, edge_attr, W_in, b_in,
           W0_1, b0_1, W0_2, b0_2, g0, be0,
           W1_1, b1_1, W1_2, b1_2, g1, be1,
           W2_1, b2_1, W2_2, b2_2, g2, be2,
           Wo1, bo1, Wo2, bo2):
    f32 = jnp.float32
    src = edge_index[0]
    dst = edge_index[1]
    pad = EP - E
    srcp = jnp.concatenate([src, jnp.zeros((pad,), jnp.int32)])
    dstp = jnp.concatenate([dst, jnp.full((pad,), TRASH, jnp.int32)])
    perm = lambda a: a.reshape(EP // CHUNK, CHUNK // 2, 2).transpose(0, 2, 1).reshape(EP // CHUNK, CHUNK)
    src2 = jnp.stack([perm(srcp), perm(srcp + N)])
    dst2d = perm(dstp)

    layers = [(W0_1, b0_1, W0_2, b0_2, g0, be0),
              (W1_1, b1_1, W1_2, b1_2, g1, be1),
              (W2_1, b2_1, W2_2, b2_2, g2, be2)]
    row = lambda v: v.reshape(1, -1)

    ea2 = edge_attr.reshape(E // 2, 32)
    zb = jnp.zeros((16, H), jnp.float32)
    bd = lambda W: jnp.concatenate(
        [jnp.concatenate([W[H:], zb], axis=1),
         jnp.concatenate([zb, W[H:]], axis=1)], axis=0)
    degp = _sc_degrees(dst2d)
    q_flats = _edge_proj(ea2, bd(W0_1), bd(W1_1), bd(W2_1))
    h, p3 = _input_proj(x, W_in, row(b_in), W0_1[:H], row(b0_1))

    for l, (Wa, ba, Wb, bb, g, be) in enumerate(layers):
        acc2 = _sc_layer(src2, dst2d, p3.reshape(2 * N, HALF), q_flats[l])
        agg, stats = _aggregate(acc2, degp, Wb, row(bb))
        if l < 2:
            Wn, ban = layers[l + 1][0], layers[l + 1][1]
            h, p3 = _update_and_proj(agg, stats, g.reshape(1, H), be.reshape(1, H),
                                     h, Wn[:H], row(ban))
        else:
            Wo2p = jnp.zeros((H, 128), f32).at[:, :3].set(Wo2)
            bo2p = jnp.zeros((1, 128), f32).at[0, :3].set(bo2)
            outp = _final_out(agg, stats, g.reshape(1, H), be.reshape(1, H),
                              h, Wo1, row(bo1), Wo2p, bo2p)
    return outp[:, :3]


# speedup vs baseline: 1.2923x; 1.2923x over previous
import functools

import jax
import jax.numpy as jnp
from jax import lax
from jax.experimental import pallas as pl
from jax.experimental.pallas import tpu as pltpu
from jax.experimental.pallas import tpu_sc as plsc

N = 50000
E = 800000
EP = 819200
CHUNK = 64
ROWS = EP // CHUNK
H = 64
HALF = 32
NC, NS, L = 2, 16, 16
NACC = 50048
TRASH = N
BLK = 1000


def _k1_body(x_ref, win_ref, bin_ref, wa_ref, ba_ref, h_ref, p3_ref):
    h = jnp.maximum(jnp.dot(x_ref[...], win_ref[...],
                            preferred_element_type=jnp.float32, precision=lax.Precision.HIGHEST) + bin_ref[...], 0.0)
    h_ref[...] = h
    p = jnp.dot(h, wa_ref[...], preferred_element_type=jnp.float32, precision=lax.Precision.HIGHEST) + ba_ref[...]
    p3_ref[...] = jnp.stack([p[:, :HALF], p[:, HALF:]], axis=0)


def _input_proj(x, W_in, b_in, Wa_h, ba):
    grid = N // BLK
    return pl.pallas_call(
        _k1_body,
        grid=(grid,),
        in_specs=[
            pl.BlockSpec((BLK, 128), lambda r: (r, 0)),
            pl.BlockSpec((128, H), lambda r: (0, 0)),
            pl.BlockSpec((1, H), lambda r: (0, 0)),
            pl.BlockSpec((H, H), lambda r: (0, 0)),
            pl.BlockSpec((1, H), lambda r: (0, 0)),
        ],
        out_specs=[
            pl.BlockSpec((BLK, H), lambda r: (r, 0)),
            pl.BlockSpec((2, BLK, HALF), lambda r: (0, r, 0)),
        ],
        out_shape=[
            jax.ShapeDtypeStruct((N, H), jnp.float32),
            jax.ShapeDtypeStruct((2, N, HALF), jnp.float32),
        ],
    )(x, W_in, b_in, Wa_h, ba)


def _kq_body(ea_ref, w0_ref, w1_ref, w2_ref, q0_ref, q1_ref, q2_ref):
    ea = ea_ref[...]
    for w, out in ((w0_ref, q0_ref), (w1_ref, q1_ref), (w2_ref, q2_ref)):
        out[...] = jnp.dot(ea, w[...], preferred_element_type=jnp.float32,
                           precision=lax.Precision.HIGHEST)


def _edge_proj(ea2, W0e2, W1e2, W2e2):
    eb = 8000
    grid = (E // 2) // eb
    return pl.pallas_call(
        _kq_body,
        grid=(grid,),
        in_specs=[
            pl.BlockSpec((eb, 32), lambda r: (r, 0)),
            pl.BlockSpec((32, 128), lambda r: (0, 0)),
            pl.BlockSpec((32, 128), lambda r: (0, 0)),
            pl.BlockSpec((32, 128), lambda r: (0, 0)),
        ],
        out_specs=[pl.BlockSpec((eb, 128), lambda r: (r, 0))] * 3,
        out_shape=[jax.ShapeDtypeStruct((E // 2, 128), jnp.float32)] * 3,
    )(ea2, W0e2, W1e2, W2e2)


def _kagg_body(acc_ref, degp_ref, wb_ref, bb_ref, agg_ref, stats_ref):
    r = pl.program_id(0)
    s = jnp.concatenate([acc_ref[0], acc_ref[1]], axis=-1)
    deg = (degp_ref[0, :, 0:1] + degp_ref[1, :, 0:1])
    a = jnp.dot(s, wb_ref[...], preferred_element_type=jnp.float32, precision=lax.Precision.HIGHEST) + deg * bb_ref[...]
    agg_ref[...] = a
    zc = jnp.zeros((1, 128 - H), jnp.float32)
    r0 = jnp.concatenate([jnp.sum(a, axis=0).reshape(1, H), zc], axis=1)
    r1 = jnp.concatenate([jnp.sum(a * a, axis=0).reshape(1, H), zc], axis=1)
    upd = jnp.concatenate([r0, r1, jnp.zeros((6, 128), jnp.float32)], axis=0)

    @pl.when(r == 0)
    def _():
        stats_ref[...] = jnp.zeros((8, 128), jnp.float32)

    stats_ref[...] += upd


def _aggregate(acc2, degp, Wb, bb):
    grid = N // BLK
    return pl.pallas_call(
        _kagg_body,
        grid=(grid,),
        in_specs=[
            pl.BlockSpec((2, BLK, HALF), lambda r: (0, r, 0)),
            pl.BlockSpec((2, BLK, 16), lambda r: (0, r, 0)),
            pl.BlockSpec((H, H), lambda r: (0, 0)),
            pl.BlockSpec((1, H), lambda r: (0, 0)),
        ],
        out_specs=[
            pl.BlockSpec((BLK, H), lambda r: (r, 0)),
            pl.BlockSpec((8, 128), lambda r: (0, 0)),
        ],
        out_shape=[
            jax.ShapeDtypeStruct((N, H), jnp.float32),
            jax.ShapeDtypeStruct((8, 128), jnp.float32),
        ],
    )(acc2, degp, Wb, bb)


def _bn_from_stats(stats_ref, agg, g_ref, be_ref, h_ref):
    mu = stats_ref[0:1, :H] * (1.0 / N)
    ex2 = stats_ref[1:2, :H] * (1.0 / N)
    var = ex2 - mu * mu
    inv = g_ref[...] * lax.rsqrt(var + 1e-5)
    return h_ref[...] + jnp.maximum((agg - mu) * inv + be_ref[...], 0.0)


def _kupd_body(agg_ref, stats_ref, g_ref, be_ref, h_ref, wa_ref, ba_ref,
               hn_ref, p3_ref):
    hn = _bn_from_stats(stats_ref, agg_ref[...], g_ref, be_ref, h_ref)
    hn_ref[...] = hn
    p = jnp.dot(hn, wa_ref[...], preferred_element_type=jnp.float32, precision=lax.Precision.HIGHEST) + ba_ref[...]
    p3_ref[...] = jnp.stack([p[:, :HALF], p[:, HALF:]], axis=0)


def _update_and_proj(agg, stats, g, be, h, Wa_h, ba):
    grid = N // BLK
    return pl.pallas_call(
        _kupd_body,
        grid=(grid,),
        in_specs=[
            pl.BlockSpec((BLK, H), lambda r: (r, 0)),
            pl.BlockSpec((8, 128), lambda r: (0, 0)),
            pl.BlockSpec((1, H), lambda r: (0, 0)),
            pl.BlockSpec((1, H), lambda r: (0, 0)),
            pl.BlockSpec((BLK, H), lambda r: (r, 0)),
            pl.BlockSpec((H, H), lambda r: (0, 0)),
            pl.BlockSpec((1, H), lambda r: (0, 0)),
        ],
        out_specs=[
            pl.BlockSpec((BLK, H), lambda r: (r, 0)),
            pl.BlockSpec((2, BLK, HALF), lambda r: (0, r, 0)),
        ],
        out_shape=[
            jax.ShapeDtypeStruct((N, H), jnp.float32),
            jax.ShapeDtypeStruct((2, N, HALF), jnp.float32),
        ],
    )(agg, stats, g, be, h, Wa_h, ba)


def _kout_body(agg_ref, stats_ref, g_ref, be_ref, h_ref,
               wo1_ref, bo1_ref, wo2_ref, bo2_ref, out_ref):
    hn = _bn_from_stats(stats_ref, agg_ref[...], g_ref, be_ref, h_ref)
    t = jnp.maximum(jnp.dot(hn, wo1_ref[...],
                            preferred_element_type=jnp.float32, precision=lax.Precision.HIGHEST) + bo1_ref[...], 0.0)
    out_ref[...] = jnp.dot(t, wo2_ref[...],
                           preferred_element_type=jnp.float32, precision=lax.Precision.HIGHEST) + bo2_ref[...]


def _final_out(agg, stats, g, be, h, Wo1, bo1, Wo2p, bo2p):
    grid = N // BLK
    return pl.pallas_call(
        _kout_body,
        grid=(grid,),
        in_specs=[
            pl.BlockSpec((BLK, H), lambda r: (r, 0)),
            pl.BlockSpec((8, 128), lambda r: (0, 0)),
            pl.BlockSpec((1, H), lambda r: (0, 0)),
            pl.BlockSpec((1, H), lambda r: (0, 0)),
            pl.BlockSpec((BLK, H), lambda r: (r, 0)),
            pl.BlockSpec((H, H), lambda r: (0, 0)),
            pl.BlockSpec((1, H), lambda r: (0, 0)),
            pl.BlockSpec((H, 128), lambda r: (0, 0)),
            pl.BlockSpec((1, 128), lambda r: (0, 0)),
        ],
        out_specs=[pl.BlockSpec((BLK, 128), lambda r: (r, 0))],
        out_shape=[jax.ShapeDtypeStruct((N, 128), jnp.float32)],
    )(agg, stats, g, be, h, Wo1, bo1, Wo2p, bo2p)[0]


_MESH = plsc.VectorSubcoreMesh(core_axis_name="c", subcore_axis_name="s",
                               num_cores=NC, num_subcores=NS)

ZROWS = 136


def _zero_spmem(zbuf, acc_sh, sid, width):
    @pl.loop(0, ZROWS)
    def _(i):
        for c2 in range(width // L):
            zbuf[i, pl.ds(c2 * L, L)] = jnp.zeros((L,), jnp.float32)

    for t in range(23):
        pltpu.sync_copy(zbuf, acc_sh.at[pl.ds(sid * 3128 + t * ZROWS, ZROWS), :])


def _writeback(acc_sh, out, cid, sid):
    nb = 3128
    start = pl.multiple_of(jnp.minimum(sid * nb, N - nb), 8)
    pltpu.sync_copy(acc_sh.at[pl.ds(start, nb), :],
                    out.at[cid, pl.ds(start, nb), :])


CPS = ROWS // NS


def _sc_layer_body(src2, dst2d, p_flat, q_flat, acc_out,
                   acc_sh, sidx, didx, rows, qv, zbuf, isem, gsem, qsem, ssem):
    cid = lax.axis_index("c")
    sid = lax.axis_index("s")
    _zero_spmem(zbuf, acc_sh, sid, HALF)
    plsc.subcore_barrier()

    row0 = sid * CPS
    qcol = cid * HALF

    def drain(src, dst, sem):
        pltpu.make_async_copy(src, dst, sem).wait()

    pltpu.async_copy(src2.at[cid, row0], sidx.at[0], isem.at[0])
    pltpu.async_copy(dst2d.at[row0], didx.at[0], isem.at[0])
    pltpu.async_copy(src2.at[cid, row0 + 1], sidx.at[1], isem.at[1])
    pltpu.async_copy(dst2d.at[row0 + 1], didx.at[1], isem.at[1])

    qdummy = q_flat.at[pl.ds(0, CHUNK), pl.ds(0, HALF)]
    gdummy = p_flat.at[pl.ds(0, CHUNK), :]
    sdummy = acc_sh.at[pl.ds(0, CHUNK), :]

    @pl.loop(0, CPS + 8, step=8)
    def _steps(t0):
        for b in range(8):
            t = t0 + b
            bd = b % 4
            bp = (b + 2) % 4
            ip = (b + 6) % 8
            inx = (b + 2) % 8

            @pl.when(t < CPS)
            def _():
                @pl.when(t >= 4)
                def _():
                    drain(gdummy, sdummy, ssem.at[bd])

                drain(src2.at[cid, 0], sidx.at[b], isem.at[b])
                drain(dst2d.at[0], didx.at[b], isem.at[b])
                pltpu.async_copy(p_flat.at[sidx.at[b]], rows.at[bd], gsem.at[bd])
                qoff = jnp.minimum((row0 + t) * 32, E // 2 - 32)
                pltpu.async_copy(
                    q_flat.at[pl.ds(qoff, 32), pl.ds(qcol, HALF)],
                    qv.at[bd, pl.ds(0, 32), :], qsem.at[bd])
                pltpu.async_copy(
                    q_flat.at[pl.ds(qoff, 32), pl.ds(64 + qcol, HALF)],
                    qv.at[bd, pl.ds(32, 32), :], qsem.at[bd])

            @pl.when(jnp.logical_and(t >= 2, t < CPS + 2))
            def _():
                drain(gdummy, rows.at[bp], gsem.at[bp])
                drain(qdummy, qv.at[bp], qsem.at[bp])

                @pl.loop(0, CHUNK, unroll=4)
                def _cmp(i):
                    qrow = 32 * (i % 2) + i // 2
                    for c2 in range(2):
                        sl = pl.ds(c2 * L, L)
                        rows[bp, i, sl] = jnp.maximum(
                            rows[bp, i, sl] + qv[bp, qrow, sl], 0.0)

                pltpu.async_copy(rows.at[bp], acc_sh.at[didx.at[ip]],
                                 ssem.at[bp], add=True)

            @pl.when(t + 2 < CPS)
            def _():
                pltpu.async_copy(src2.at[cid, row0 + t + 2], sidx.at[inx],
                                 isem.at[inx])
                pltpu.async_copy(dst2d.at[row0 + t + 2], didx.at[inx],
                                 isem.at[inx])

    for slot in range(4):
        drain(gdummy, sdummy, ssem.at[slot])
    plsc.subcore_barrier()
    _writeback(acc_sh, acc_out, cid, sid)


def _sc_layer(src2, dst2d, p_flat, q_flat):
    return pl.kernel(
        _sc_layer_body,
        out_type=jax.ShapeDtypeStruct((2, N, HALF), jnp.float32),
        mesh=_MESH,
        scratch_types=[
            pltpu.VMEM_SHARED((NACC, HALF), jnp.float32),
            pltpu.VMEM((8, CHUNK), jnp.int32),
            pltpu.VMEM((8, CHUNK), jnp.int32),
            pltpu.VMEM((4, CHUNK, HALF), jnp.float32),
            pltpu.VMEM((4, CHUNK, HALF), jnp.float32),
            pltpu.VMEM((ZROWS, HALF), jnp.float32),
            pltpu.SemaphoreType.DMA((8,)),
            pltpu.SemaphoreType.DMA((4,)),
            pltpu.SemaphoreType.DMA((4,)),
            pltpu.SemaphoreType.DMA((4,)),
        ],
        compiler_params=pltpu.CompilerParams(use_tc_tiling_on_sc=False),
    )(src2, dst2d, p_flat, q_flat)


def _sc_deg_body(dst2d, deg_out, acc_sh, didx, ones, zbuf):
    cid = lax.axis_index("c")
    sid = lax.axis_index("s")
    _zero_spmem(zbuf, acc_sh, sid, 16)

    @pl.loop(0, CHUNK)
    def _(i):
        ones[i, pl.ds(0, L)] = jnp.full((L,), 1.0, jnp.float32)

    plsc.subcore_barrier()

    wid = cid * NS + sid
    row0 = wid * (ROWS // (NC * NS))

    @pl.loop(0, ROWS // (NC * NS) // 16)
    def _outer(o):
        r0 = row0 + o * 16
        pltpu.sync_copy(dst2d.at[pl.ds(r0, 16), :], didx)

        @pl.loop(0, 16)
        def _chunk(j):
            pltpu.sync_copy(ones, acc_sh.at[didx.at[j]], add=True)

    plsc.subcore_barrier()
    _writeback(acc_sh, deg_out, cid, sid)


def _sc_degrees(dst2d):
    return pl.kernel(
        _sc_deg_body,
        out_type=jax.ShapeDtypeStruct((2, N, 16), jnp.float32),
        mesh=_MESH,
        scratch_types=[
            pltpu.VMEM_SHARED((NACC, 16), jnp.float32),
            pltpu.VMEM((16, CHUNK), jnp.int32),
            pltpu.VMEM((CHUNK, 16), jnp.float32),
            pltpu.VMEM((ZROWS, 16), jnp.float32),
        ],
        compiler_params=pltpu.CompilerParams(use_tc_tiling_on_sc=False),
    )(dst2d)


def kernel(x, edge_index, edge_attr, W_in, b_in,
           W0_1, b0_1, W0_2, b0_2, g0, be0,
           W1_1, b1_1, W1_2, b1_2, g1, be1,
           W2_1, b2_1, W2_2, b2_2, g2, be2,
           Wo1, bo1, Wo2, bo2):
    f32 = jnp.float32
    src = edge_index[0]
    dst = edge_index[1]
    pad = EP - E
    srcp = jnp.concatenate([src, jnp.zeros((pad,), jnp.int32)])
    dstp = jnp.concatenate([dst, jnp.full((pad,), TRASH, jnp.int32)])
    src2 = jnp.stack([srcp, srcp + N]).reshape(2, ROWS, CHUNK)
    dst2d = dstp.reshape(ROWS, CHUNK)

    layers = [(W0_1, b0_1, W0_2, b0_2, g0, be0),
              (W1_1, b1_1, W1_2, b1_2, g1, be1),
              (W2_1, b2_1, W2_2, b2_2, g2, be2)]
    row = lambda v: v.reshape(1, -1)

    ea2 = edge_attr.reshape(E // 2, 32)
    zb = jnp.zeros((16, H), jnp.float32)
    bd = lambda W: jnp.concatenate(
        [jnp.concatenate([W[H:], zb], axis=1),
         jnp.concatenate([zb, W[H:]], axis=1)], axis=0)
    degp = _sc_degrees(dst2d)
    q_flats = _edge_proj(ea2, bd(W0_1), bd(W1_1), bd(W2_1))
    h, p3 = _input_proj(x, W_in, row(b_in), W0_1[:H], row(b0_1))

    for l, (Wa, ba, Wb, bb, g, be) in enumerate(layers):
        acc2 = _sc_layer(src2, dst2d, p3.reshape(2 * N, HALF), q_flats[l])
        agg, stats = _aggregate(acc2, degp, Wb, row(bb))
        if l < 2:
            Wn, ban = layers[l + 1][0], layers[l + 1][1]
            h, p3 = _update_and_proj(agg, stats, g.reshape(1, H), be.reshape(1, H),
                                     h, Wn[:H], row(ban))
        else:
            Wo2p = jnp.zeros((H, 128), f32).at[:, :3].set(Wo2)
            bo2p = jnp.zeros((1, 128), f32).at[0, :3].set(bo2)
            outp = _final_out(agg, stats, g.reshape(1, H), be.reshape(1, H),
                              h, Wo1, row(bo1), Wo2p, bo2p)
    return outp[:, :3]


# speedup vs baseline: 1.3227x; 1.0236x over previous
import functools

import jax
import jax.numpy as jnp
from jax import lax
from jax.experimental import pallas as pl
from jax.experimental.pallas import tpu as pltpu
from jax.experimental.pallas import tpu_sc as plsc

N = 50000
E = 800000
EP = 819200
CHUNK = 64
ROWS = EP // CHUNK
H = 64
HALF = 32
NC, NS, L = 2, 16, 16
NACC = 50048
TRASH = N
BLK = 1000


def _k1_body(x_ref, win_ref, bin_ref, wa_ref, ba_ref, h_ref, p3_ref):
    h = jnp.maximum(jnp.dot(x_ref[...], win_ref[...],
                            preferred_element_type=jnp.float32, precision=lax.Precision.HIGHEST) + bin_ref[...], 0.0)
    h_ref[...] = h
    p = jnp.dot(h, wa_ref[...], preferred_element_type=jnp.float32, precision=lax.Precision.HIGHEST) + ba_ref[...]
    p3_ref[...] = jnp.stack([p[:, :HALF], p[:, HALF:]], axis=0)


def _input_proj(x, W_in, b_in, Wa_h, ba):
    grid = N // BLK
    return pl.pallas_call(
        _k1_body,
        grid=(grid,),
        in_specs=[
            pl.BlockSpec((BLK, 128), lambda r: (r, 0)),
            pl.BlockSpec((128, H), lambda r: (0, 0)),
            pl.BlockSpec((1, H), lambda r: (0, 0)),
            pl.BlockSpec((H, H), lambda r: (0, 0)),
            pl.BlockSpec((1, H), lambda r: (0, 0)),
        ],
        out_specs=[
            pl.BlockSpec((BLK, H), lambda r: (r, 0)),
            pl.BlockSpec((2, BLK, HALF), lambda r: (0, r, 0)),
        ],
        out_shape=[
            jax.ShapeDtypeStruct((N, H), jnp.float32),
            jax.ShapeDtypeStruct((2, N, HALF), jnp.float32),
        ],
    )(x, W_in, b_in, Wa_h, ba)


def _kq_body(ea_ref, w_ref, q_ref):
    q_ref[...] = jnp.dot(ea_ref[...], w_ref[...],
                         preferred_element_type=jnp.float32,
                         precision=lax.Precision.HIGHEST)


def _edge_proj(ea2, We2):
    eb = 8000
    grid = (E // 2) // eb
    return pl.pallas_call(
        _kq_body,
        grid=(grid,),
        in_specs=[
            pl.BlockSpec((eb, 32), lambda r: (r, 0)),
            pl.BlockSpec((32, 128), lambda r: (0, 0)),
        ],
        out_specs=[pl.BlockSpec((eb, 128), lambda r: (r, 0))],
        out_shape=[jax.ShapeDtypeStruct((E // 2, 128), jnp.float32)],
    )(ea2, We2)[0]


def _kagg_body(acc_ref, degp_ref, wb_ref, bb_ref, agg_ref, stats_ref):
    r = pl.program_id(0)
    s = jnp.concatenate([acc_ref[0], acc_ref[1]], axis=-1)
    deg = (degp_ref[0, :, 0:1] + degp_ref[1, :, 0:1])
    a = jnp.dot(s, wb_ref[...], preferred_element_type=jnp.float32, precision=lax.Precision.HIGHEST) + deg * bb_ref[...]
    agg_ref[...] = a
    zc = jnp.zeros((1, 128 - H), jnp.float32)
    r0 = jnp.concatenate([jnp.sum(a, axis=0).reshape(1, H), zc], axis=1)
    r1 = jnp.concatenate([jnp.sum(a * a, axis=0).reshape(1, H), zc], axis=1)
    upd = jnp.concatenate([r0, r1, jnp.zeros((6, 128), jnp.float32)], axis=0)

    @pl.when(r == 0)
    def _():
        stats_ref[...] = jnp.zeros((8, 128), jnp.float32)

    stats_ref[...] += upd


def _aggregate(acc2, degp, Wb, bb):
    grid = N // BLK
    return pl.pallas_call(
        _kagg_body,
        grid=(grid,),
        in_specs=[
            pl.BlockSpec((2, BLK, HALF), lambda r: (0, r, 0)),
            pl.BlockSpec((2, BLK, 16), lambda r: (0, r, 0)),
            pl.BlockSpec((H, H), lambda r: (0, 0)),
            pl.BlockSpec((1, H), lambda r: (0, 0)),
        ],
        out_specs=[
            pl.BlockSpec((BLK, H), lambda r: (r, 0)),
            pl.BlockSpec((8, 128), lambda r: (0, 0)),
        ],
        out_shape=[
            jax.ShapeDtypeStruct((N, H), jnp.float32),
            jax.ShapeDtypeStruct((8, 128), jnp.float32),
        ],
    )(acc2, degp, Wb, bb)


def _bn_from_stats(stats_ref, agg, g_ref, be_ref, h_ref):
    mu = stats_ref[0:1, :H] * (1.0 / N)
    ex2 = stats_ref[1:2, :H] * (1.0 / N)
    var = ex2 - mu * mu
    inv = g_ref[...] * lax.rsqrt(var + 1e-5)
    return h_ref[...] + jnp.maximum((agg - mu) * inv + be_ref[...], 0.0)


def _kupd_body(agg_ref, stats_ref, g_ref, be_ref, h_ref, wa_ref, ba_ref,
               hn_ref, p3_ref):
    hn = _bn_from_stats(stats_ref, agg_ref[...], g_ref, be_ref, h_ref)
    hn_ref[...] = hn
    p = jnp.dot(hn, wa_ref[...], preferred_element_type=jnp.float32, precision=lax.Precision.HIGHEST) + ba_ref[...]
    p3_ref[...] = jnp.stack([p[:, :HALF], p[:, HALF:]], axis=0)


def _update_and_proj(agg, stats, g, be, h, Wa_h, ba):
    grid = N // BLK
    return pl.pallas_call(
        _kupd_body,
        grid=(grid,),
        in_specs=[
            pl.BlockSpec((BLK, H), lambda r: (r, 0)),
            pl.BlockSpec((8, 128), lambda r: (0, 0)),
            pl.BlockSpec((1, H), lambda r: (0, 0)),
            pl.BlockSpec((1, H), lambda r: (0, 0)),
            pl.BlockSpec((BLK, H), lambda r: (r, 0)),
            pl.BlockSpec((H, H), lambda r: (0, 0)),
            pl.BlockSpec((1, H), lambda r: (0, 0)),
        ],
        out_specs=[
            pl.BlockSpec((BLK, H), lambda r: (r, 0)),
            pl.BlockSpec((2, BLK, HALF), lambda r: (0, r, 0)),
        ],
        out_shape=[
            jax.ShapeDtypeStruct((N, H), jnp.float32),
            jax.ShapeDtypeStruct((2, N, HALF), jnp.float32),
        ],
    )(agg, stats, g, be, h, Wa_h, ba)


def _kout_body(agg_ref, stats_ref, g_ref, be_ref, h_ref,
               wo1_ref, bo1_ref, wo2_ref, bo2_ref, out_ref):
    hn = _bn_from_stats(stats_ref, agg_ref[...], g_ref, be_ref, h_ref)
    t = jnp.maximum(jnp.dot(hn, wo1_ref[...],
                            preferred_element_type=jnp.float32, precision=lax.Precision.HIGHEST) + bo1_ref[...], 0.0)
    out_ref[...] = jnp.dot(t, wo2_ref[...],
                           preferred_element_type=jnp.float32, precision=lax.Precision.HIGHEST) + bo2_ref[...]


def _final_out(agg, stats, g, be, h, Wo1, bo1, Wo2p, bo2p):
    grid = N // BLK
    return pl.pallas_call(
        _kout_body,
        grid=(grid,),
        in_specs=[
            pl.BlockSpec((BLK, H), lambda r: (r, 0)),
            pl.BlockSpec((8, 128), lambda r: (0, 0)),
            pl.BlockSpec((1, H), lambda r: (0, 0)),
            pl.BlockSpec((1, H), lambda r: (0, 0)),
            pl.BlockSpec((BLK, H), lambda r: (r, 0)),
            pl.BlockSpec((H, H), lambda r: (0, 0)),
            pl.BlockSpec((1, H), lambda r: (0, 0)),
            pl.BlockSpec((H, 128), lambda r: (0, 0)),
            pl.BlockSpec((1, 128), lambda r: (0, 0)),
        ],
        out_specs=[pl.BlockSpec((BLK, 128), lambda r: (r, 0))],
        out_shape=[jax.ShapeDtypeStruct((N, 128), jnp.float32)],
    )(agg, stats, g, be, h, Wo1, bo1, Wo2p, bo2p)[0]


_MESH = plsc.VectorSubcoreMesh(core_axis_name="c", subcore_axis_name="s",
                               num_cores=NC, num_subcores=NS)

ZROWS = 136


def _zero_spmem(zbuf, acc_sh, sid, width):
    @pl.loop(0, ZROWS)
    def _(i):
        for c2 in range(width // L):
            zbuf[i, pl.ds(c2 * L, L)] = jnp.zeros((L,), jnp.float32)

    for t in range(23):
        pltpu.sync_copy(zbuf, acc_sh.at[pl.ds(sid * 3128 + t * ZROWS, ZROWS), :])


def _writeback(acc_sh, out, cid, sid):
    nb = 3128
    start = pl.multiple_of(jnp.minimum(sid * nb, N - nb), 8)
    pltpu.sync_copy(acc_sh.at[pl.ds(start, nb), :],
                    out.at[cid, pl.ds(start, nb), :])


CPS = ROWS // NS


def _sc_layer_body(src2, dst2d, p_flat, q_flat, acc_out,
                   acc_sh, sidx, didx, rows, qv, zbuf, isem, gsem, qsem, ssem):
    cid = lax.axis_index("c")
    sid = lax.axis_index("s")
    _zero_spmem(zbuf, acc_sh, sid, HALF)
    plsc.subcore_barrier()

    row0 = sid * CPS
    qcol = cid * HALF

    def drain(src, dst, sem):
        pltpu.make_async_copy(src, dst, sem).wait()

    pltpu.async_copy(src2.at[cid, row0], sidx.at[0], isem.at[0])
    pltpu.async_copy(dst2d.at[row0], didx.at[0], isem.at[0])
    pltpu.async_copy(src2.at[cid, row0 + 1], sidx.at[1], isem.at[1])
    pltpu.async_copy(dst2d.at[row0 + 1], didx.at[1], isem.at[1])

    qdummy = q_flat.at[pl.ds(0, CHUNK), pl.ds(0, HALF)]
    gdummy = p_flat.at[pl.ds(0, CHUNK), :]
    sdummy = acc_sh.at[pl.ds(0, CHUNK), :]

    @pl.loop(0, CPS + 8, step=8)
    def _steps(t0):
        for b in range(8):
            t = t0 + b
            bd = b % 4
            bp = (b + 2) % 4
            ip = (b + 6) % 8
            inx = (b + 2) % 8

            @pl.when(t < CPS)
            def _():
                @pl.when(t >= 4)
                def _():
                    drain(gdummy, sdummy, ssem.at[bd])

                drain(src2.at[cid, 0], sidx.at[b], isem.at[b])
                drain(dst2d.at[0], didx.at[b], isem.at[b])
                pltpu.async_copy(p_flat.at[sidx.at[b]], rows.at[bd], gsem.at[bd])
                qoff = jnp.minimum((row0 + t) * 32, E // 2 - 32)
                pltpu.async_copy(
                    q_flat.at[pl.ds(qoff, 32), pl.ds(qcol, HALF)],
                    qv.at[bd, pl.ds(0, 32), :], qsem.at[bd])
                pltpu.async_copy(
                    q_flat.at[pl.ds(qoff, 32), pl.ds(64 + qcol, HALF)],
                    qv.at[bd, pl.ds(32, 32), :], qsem.at[bd])

            @pl.when(jnp.logical_and(t >= 2, t < CPS + 2))
            def _():
                drain(gdummy, rows.at[bp], gsem.at[bp])
                drain(qdummy, qv.at[bp], qsem.at[bp])

                @pl.loop(0, CHUNK, unroll=4)
                def _cmp(i):
                    qrow = 32 * (i % 2) + i // 2
                    for c2 in range(2):
                        sl = pl.ds(c2 * L, L)
                        rows[bp, i, sl] = jnp.maximum(
                            rows[bp, i, sl] + qv[bp, qrow, sl], 0.0)

                pltpu.async_copy(rows.at[bp], acc_sh.at[didx.at[ip]],
                                 ssem.at[bp], add=True)

            @pl.when(t + 2 < CPS)
            def _():
                pltpu.async_copy(src2.at[cid, row0 + t + 2], sidx.at[inx],
                                 isem.at[inx])
                pltpu.async_copy(dst2d.at[row0 + t + 2], didx.at[inx],
                                 isem.at[inx])

    for slot in range(4):
        drain(gdummy, sdummy, ssem.at[slot])
    plsc.subcore_barrier()
    _writeback(acc_sh, acc_out, cid, sid)


def _sc_layer(src2, dst2d, p_flat, q_flat):
    return pl.kernel(
        _sc_layer_body,
        out_type=jax.ShapeDtypeStruct((2, N, HALF), jnp.float32),
        mesh=_MESH,
        scratch_types=[
            pltpu.VMEM_SHARED((NACC, HALF), jnp.float32),
            pltpu.VMEM((8, CHUNK), jnp.int32),
            pltpu.VMEM((8, CHUNK), jnp.int32),
            pltpu.VMEM((4, CHUNK, HALF), jnp.float32),
            pltpu.VMEM((4, CHUNK, HALF), jnp.float32),
            pltpu.VMEM((ZROWS, HALF), jnp.float32),
            pltpu.SemaphoreType.DMA((8,)),
            pltpu.SemaphoreType.DMA((4,)),
            pltpu.SemaphoreType.DMA((4,)),
            pltpu.SemaphoreType.DMA((4,)),
        ],
        compiler_params=pltpu.CompilerParams(use_tc_tiling_on_sc=False),
    )(src2, dst2d, p_flat, q_flat)


def _sc_deg_body(dst2d, deg_out, acc_sh, didx, ones, zbuf):
    cid = lax.axis_index("c")
    sid = lax.axis_index("s")
    _zero_spmem(zbuf, acc_sh, sid, 16)

    @pl.loop(0, CHUNK)
    def _(i):
        ones[i, pl.ds(0, L)] = jnp.full((L,), 1.0, jnp.float32)

    plsc.subcore_barrier()

    wid = cid * NS + sid
    row0 = wid * (ROWS // (NC * NS))

    @pl.loop(0, ROWS // (NC * NS) // 16)
    def _outer(o):
        r0 = row0 + o * 16
        pltpu.sync_copy(dst2d.at[pl.ds(r0, 16), :], didx)

        @pl.loop(0, 16)
        def _chunk(j):
            pltpu.sync_copy(ones, acc_sh.at[didx.at[j]], add=True)

    plsc.subcore_barrier()
    _writeback(acc_sh, deg_out, cid, sid)


def _sc_degrees(dst2d):
    return pl.kernel(
        _sc_deg_body,
        out_type=jax.ShapeDtypeStruct((2, N, 16), jnp.float32),
        mesh=_MESH,
        scratch_types=[
            pltpu.VMEM_SHARED((NACC, 16), jnp.float32),
            pltpu.VMEM((16, CHUNK), jnp.int32),
            pltpu.VMEM((CHUNK, 16), jnp.float32),
            pltpu.VMEM((ZROWS, 16), jnp.float32),
        ],
        compiler_params=pltpu.CompilerParams(use_tc_tiling_on_sc=False),
    )(dst2d)


def kernel(x, edge_index, edge_attr, W_in, b_in,
           W0_1, b0_1, W0_2, b0_2, g0, be0,
           W1_1, b1_1, W1_2, b1_2, g1, be1,
           W2_1, b2_1, W2_2, b2_2, g2, be2,
           Wo1, bo1, Wo2, bo2):
    f32 = jnp.float32
    src = edge_index[0]
    dst = edge_index[1]
    pad = EP - E
    srcp = jnp.concatenate([src, jnp.zeros((pad,), jnp.int32)])
    dstp = jnp.concatenate([dst, jnp.full((pad,), TRASH, jnp.int32)])
    src2 = jnp.stack([srcp, srcp + N]).reshape(2, ROWS, CHUNK)
    dst2d = dstp.reshape(ROWS, CHUNK)

    layers = [(W0_1, b0_1, W0_2, b0_2, g0, be0),
              (W1_1, b1_1, W1_2, b1_2, g1, be1),
              (W2_1, b2_1, W2_2, b2_2, g2, be2)]
    row = lambda v: v.reshape(1, -1)

    ea2 = edge_attr.reshape(E // 2, 32)
    zb = jnp.zeros((16, H), jnp.float32)
    bd = lambda W: jnp.concatenate(
        [jnp.concatenate([W[H:], zb], axis=1),
         jnp.concatenate([zb, W[H:]], axis=1)], axis=0)
    degp = _sc_degrees(dst2d)
    q_flats = [_edge_proj(ea2, bd(W)) for W in (W0_1, W1_1, W2_1)]
    h, p3 = _input_proj(x, W_in, row(b_in), W0_1[:H], row(b0_1))

    for l, (Wa, ba, Wb, bb, g, be) in enumerate(layers):
        acc2 = _sc_layer(src2, dst2d, p3.reshape(2 * N, HALF), q_flats[l])
        agg, stats = _aggregate(acc2, degp, Wb, row(bb))
        if l < 2:
            Wn, ban = layers[l + 1][0], layers[l + 1][1]
            h, p3 = _update_and_proj(agg, stats, g.reshape(1, H), be.reshape(1, H),
                                     h, Wn[:H], row(ban))
        else:
            Wo2p = jnp.zeros((H, 128), f32).at[:, :3].set(Wo2)
            bo2p = jnp.zeros((1, 128), f32).at[0, :3].set(bo2)
            outp = _final_out(agg, stats, g.reshape(1, H), be.reshape(1, H),
                              h, Wo1, row(bo1), Wo2p, bo2p)
    return outp[:, :3]
